# Initial kernel scaffold; baseline (speedup 1.0000x reference)
import jax
import jax.numpy as jnp
from jax.experimental import pallas as pl


def kernel(x, edge_index, batch, W_in, b_in, g0, be0, W1, b1, g1, be1, W2, b2, g2, be2, W3, b3, g3, be3, P1W, P1b, P2W, P2b, P3W, P3b):
    raise NotImplementedError("write your pallas kernel here")


# speedup vs baseline: 6.8252x; 6.8252x over previous
import functools

import jax
import jax.numpy as jnp
from jax import lax
from jax.experimental import pallas as pl
from jax.experimental.pallas import tpu as pltpu
from jax.experimental.pallas import tpu_sc as plsc

N = 10000
E = 160000
NG = 64
CH = 128
RB = 1000
NB = N // RB
NSC = 16
SUB_STRIDE = 624
SUB_LEN = 640
EDGES_PER_SUB = E // NSC
EB = 80
DEG_EB = 40
EDGES_PER_WORKER = E // 32

_F32 = jnp.float32
_PREC = lax.Precision.HIGHEST

_sc_mesh = dict(core_axis_name="c", subcore_axis_name="s",
                num_cores=2, num_subcores=NSC)


_SC_CACHE = {}


def _get_deg_sc():
    if "deg" in _SC_CACHE:
        return _SC_CACHE["deg"]

    @functools.partial(
        pl.kernel,
        out_type=jax.ShapeDtypeStruct((2, N, CH), _F32),
        mesh=plsc.VectorSubcoreMesh(**_sc_mesh),
        scratch_types=[
            pltpu.VMEM((DEG_EB,), jnp.int32),
            pltpu.VMEM((DEG_EB, CH), _F32),
            pltpu.VMEM_SHARED((N, CH), _F32),
        ],
    )
    def _deg_sc(dst_hbm, zeros_hbm, ones_hbm, out_hbm, idx_v, ones_v, deg_sh):
        c = lax.axis_index("c")
        s = lax.axis_index("s")
        rs = pl.ds(s * SUB_STRIDE, SUB_LEN)
        pltpu.sync_copy(zeros_hbm.at[rs], deg_sh.at[rs])
        pltpu.sync_copy(ones_hbm, ones_v)
        plsc.subcore_barrier()
        base = (c * NSC + s) * EDGES_PER_WORKER

        def body(i, carry):
            pltpu.sync_copy(dst_hbm.at[pl.ds(base + i * DEG_EB, DEG_EB)], idx_v)
            pltpu.sync_copy(ones_v, deg_sh.at[idx_v], add=True)
            return carry

        lax.fori_loop(0, EDGES_PER_WORKER // DEG_EB, body, 0)
        plsc.subcore_barrier()
        pltpu.sync_copy(deg_sh.at[rs], out_hbm.at[c, rs])

    _SC_CACHE["deg"] = _deg_sc
    return _deg_sc


def _get_agg(nchunk):
    key = ("agg", nchunk)
    if key in _SC_CACHE:
        return _SC_CACHE[key]
    outs = [jax.ShapeDtypeStruct((N, CH), _F32) for _ in range(nchunk)]
    scratch = [
        pltpu.VMEM((EB,), jnp.int32),
        pltpu.VMEM((EB,), jnp.int32),
        pltpu.VMEM((EB, CH), _F32),
        pltpu.VMEM_SHARED((N, CH), _F32),
        pltpu.SemaphoreType.DMA,
    ]

    @functools.partial(
        pl.kernel,
        out_type=outs,
        mesh=plsc.VectorSubcoreMesh(**_sc_mesh),
        scratch_types=scratch,
    )
    def agg(src_hbm, dst_hbm, *rest):
        ps = rest[:nchunk]
        os_ = rest[nchunk:2 * nchunk]
        sidx, didx, rows, acc, sem = rest[2 * nchunk:]
        c = lax.axis_index("c")
        s = lax.axis_index("s")
        rs = pl.ds(s * SUB_STRIDE, SUB_LEN)
        for j in range(nchunk):
            p, o = ps[j], os_[j]

            @pl.when(c == (j % 2))
            def _process(p=p, o=o):
                pltpu.sync_copy(p.at[rs], acc.at[rs])
                plsc.subcore_barrier()
                base = s * EDGES_PER_SUB

                def body(i, carry):
                    off = pl.ds(base + i * EB, EB)
                    pltpu.sync_copy(src_hbm.at[off], sidx)
                    pltpu.async_copy(p.at[sidx], rows, sem).wait()
                    pltpu.sync_copy(dst_hbm.at[off], didx)
                    pltpu.sync_copy(rows, acc.at[didx], add=True)
                    return carry

                lax.fori_loop(0, EDGES_PER_SUB // EB, body, 0)
                plsc.subcore_barrier()
                pltpu.sync_copy(acc.at[rs], o.at[rs])
                plsc.subcore_barrier()

    _SC_CACHE[key] = agg
    return agg


def _dinv(degp):
    def body(d_ref, o_ref):
        deg = 1.0 + d_ref[0, :, 0:1] + d_ref[1, :, 0:1]
        o_ref[...] = lax.rsqrt(deg)

    return pl.pallas_call(
        body,
        grid=(NB,),
        in_specs=[pl.BlockSpec((2, RB, CH), lambda i: (0, i, 0))],
        out_specs=pl.BlockSpec((RB, 1), lambda i: (i, 0)),
        out_shape=jax.ShapeDtypeStruct((N, 1), _F32),
    )(degp)


def _mm0(x, W, b):
    K, H = W.shape

    def body(x_ref, w_ref, b_ref, z_ref, sum_ref, sq_ref, acc_s, acc_q):
        i = pl.program_id(0)
        z = lax.dot_general(x_ref[...], w_ref[...], (((1,), (0,)), ((), ())),
                            precision=_PREC, preferred_element_type=_F32)
        z = z + b_ref[...]
        z_ref[...] = z

        @pl.when(i == 0)
        def _():
            acc_s[...] = jnp.zeros_like(acc_s)
            acc_q[...] = jnp.zeros_like(acc_q)

        acc_s[...] += jnp.sum(z, 0, keepdims=True)
        acc_q[...] += jnp.sum(z * z, 0, keepdims=True)

        @pl.when(i == NB - 1)
        def _():
            sum_ref[...] = acc_s[...]
            sq_ref[...] = acc_q[...]

    return pl.pallas_call(
        body,
        grid=(NB,),
        in_specs=[pl.BlockSpec((RB, K), lambda i: (i, 0)),
                  pl.BlockSpec((K, H), lambda i: (0, 0)),
                  pl.BlockSpec((1, H), lambda i: (0, 0))],
        out_specs=[pl.BlockSpec((RB, H), lambda i: (i, 0)),
                   pl.BlockSpec((1, H), lambda i: (0, 0)),
                   pl.BlockSpec((1, H), lambda i: (0, 0))],
        out_shape=[jax.ShapeDtypeStruct((N, H), _F32),
                   jax.ShapeDtypeStruct((1, H), _F32),
                   jax.ShapeDtypeStruct((1, H), _F32)],
        scratch_shapes=[pltpu.VMEM((1, H), _F32), pltpu.VMEM((1, H), _F32)],
    )(x, W, b)


def _mm_agg(a_list, dinv_col, W, b):
    nchunk = len(a_list)
    H = W.shape[1]

    def body(*refs):
        a_refs = refs[:nchunk]
        dinv_ref, w_ref, b_ref = refs[nchunk:nchunk + 3]
        z_ref, sum_ref, sq_ref, acc_s, acc_q = refs[nchunk + 3:]
        i = pl.program_id(0)
        dinv = dinv_ref[...]
        acc = None
        for c in range(nchunk):
            ah = a_refs[c][...] * dinv
            part = lax.dot_general(ah, w_ref[pl.ds(c * CH, CH), :],
                                   (((1,), (0,)), ((), ())),
                                   precision=_PREC, preferred_element_type=_F32)
            acc = part if acc is None else acc + part
        z = acc + b_ref[...]
        z_ref[...] = z

        @pl.when(i == 0)
        def _():
            acc_s[...] = jnp.zeros_like(acc_s)
            acc_q[...] = jnp.zeros_like(acc_q)

        acc_s[...] += jnp.sum(z, 0, keepdims=True)
        acc_q[...] += jnp.sum(z * z, 0, keepdims=True)

        @pl.when(i == NB - 1)
        def _():
            sum_ref[...] = acc_s[...]
            sq_ref[...] = acc_q[...]

    return pl.pallas_call(
        body,
        grid=(NB,),
        in_specs=[pl.BlockSpec((RB, CH), lambda i: (i, 0))
                  for _ in range(nchunk)] +
                 [pl.BlockSpec((RB, 1), lambda i: (i, 0)),
                  pl.BlockSpec(W.shape, lambda i: (0, 0)),
                  pl.BlockSpec((1, H), lambda i: (0, 0))],
        out_specs=[pl.BlockSpec((RB, H), lambda i: (i, 0)),
                   pl.BlockSpec((1, H), lambda i: (0, 0)),
                   pl.BlockSpec((1, H), lambda i: (0, 0))],
        out_shape=[jax.ShapeDtypeStruct((N, H), _F32),
                   jax.ShapeDtypeStruct((1, H), _F32),
                   jax.ShapeDtypeStruct((1, H), _F32)],
        scratch_shapes=[pltpu.VMEM((1, H), _F32), pltpu.VMEM((1, H), _F32)],
    )(*a_list, dinv_col, W, b)


def _apply(z, ssum, ssq, gamma, beta, dinv_col, nchunk):
    H = z.shape[1]

    def body(z_ref, s_ref, q_ref, g_ref, be_ref, dinv_ref, *outs):
        m = s_ref[...] / N
        v = q_ref[...] / N - m * m
        y = (z_ref[...] - m) * lax.rsqrt(v + 1e-5) * g_ref[...] + be_ref[...]
        h = jnp.maximum(y, 0.0)
        p = h * dinv_ref[...]
        for c in range(nchunk):
            outs[c][...] = p[:, c * CH:(c + 1) * CH]

    return pl.pallas_call(
        body,
        grid=(NB,),
        in_specs=[pl.BlockSpec((RB, H), lambda i: (i, 0)),
                  pl.BlockSpec((1, H), lambda i: (0, 0)),
                  pl.BlockSpec((1, H), lambda i: (0, 0)),
                  pl.BlockSpec((1, H), lambda i: (0, 0)),
                  pl.BlockSpec((1, H), lambda i: (0, 0)),
                  pl.BlockSpec((RB, 1), lambda i: (i, 0))],
        out_specs=[pl.BlockSpec((RB, CH), lambda i: (i, 0))] * nchunk,
        out_shape=[jax.ShapeDtypeStruct((N, CH), _F32)] * nchunk,
    )(z, ssum, ssq, gamma, beta, dinv_col)


def _pool(z3, s3, q3, g3, be3, batch_col, P1W, P1b, P2W, P2b, P3W, P3b):
    H = z3.shape[1]

    def body(z_ref, s_ref, q_ref, g_ref, be_ref, b_ref,
             p1w, p1b, p2w, p2b, p3w, p3b, out_ref, ssum, smax, scnt):
        i = pl.program_id(0)

        @pl.when(i == 0)
        def _():
            ssum[...] = jnp.zeros_like(ssum)
            scnt[...] = jnp.zeros_like(scnt)
            smax[...] = jnp.full_like(smax, -jnp.inf)

        m = s_ref[...] / N
        v = q_ref[...] / N - m * m
        h = jnp.maximum(
            (z_ref[...] - m) * lax.rsqrt(v + 1e-5) * g_ref[...] + be_ref[...],
            0.0)
        btc = b_ref[...]
        oh = (btc == lax.broadcasted_iota(jnp.int32, (1, NG), 1)).astype(_F32)
        ssum[...] += lax.dot_general(oh, h, (((0,), (0,)), ((), ())),
                                     precision=_PREC, preferred_element_type=_F32)
        scnt[...] += lax.dot_general(oh, jnp.ones((RB, 1), _F32),
                                     (((0,), (0,)), ((), ())),
                                     precision=_PREC, preferred_element_type=_F32)
        g_lo = jnp.min(btc)
        g_hi = jnp.max(btc)

        def mbody(g, carry):
            cur = jnp.max(jnp.where(btc == g, h, -jnp.inf), axis=0,
                          keepdims=True)
            smax[pl.ds(g, 1), :] = jnp.maximum(smax[pl.ds(g, 1), :], cur)
            return carry

        lax.fori_loop(g_lo, g_hi + 1, mbody, 0)

        @pl.when(i == NB - 1)
        def _():
            s = ssum[...]
            cnt = jnp.maximum(scnt[...], 1.0)
            mean = s / cnt
            pooled = jnp.concatenate([mean, smax[...], s], axis=1)
            h1 = jnp.maximum(
                lax.dot_general(pooled, p1w[...], (((1,), (0,)), ((), ())),
                                precision=_PREC, preferred_element_type=_F32)
                + p1b[...], 0.0)
            h2 = jnp.maximum(
                lax.dot_general(h1, p2w[...], (((1,), (0,)), ((), ())),
                                precision=_PREC, preferred_element_type=_F32)
                + p2b[...], 0.0)
            out_ref[...] = lax.dot_general(
                h2, p3w[...], (((1,), (0,)), ((), ())),
                precision=_PREC, preferred_element_type=_F32) + p3b[...]

    return pl.pallas_call(
        body,
        grid=(NB,),
        in_specs=[pl.BlockSpec((RB, H), lambda i: (i, 0)),
                  pl.BlockSpec((1, H), lambda i: (0, 0)),
                  pl.BlockSpec((1, H), lambda i: (0, 0)),
                  pl.BlockSpec((1, H), lambda i: (0, 0)),
                  pl.BlockSpec((1, H), lambda i: (0, 0)),
                  pl.BlockSpec((RB, 1), lambda i: (i, 0)),
                  pl.BlockSpec(P1W.shape, lambda i: (0, 0)),
                  pl.BlockSpec((1, P1W.shape[1]), lambda i: (0, 0)),
                  pl.BlockSpec(P2W.shape, lambda i: (0, 0)),
                  pl.BlockSpec((1, P2W.shape[1]), lambda i: (0, 0)),
                  pl.BlockSpec(P3W.shape, lambda i: (0, 0)),
                  pl.BlockSpec((1, 1), lambda i: (0, 0))],
        out_specs=pl.BlockSpec((NG, 1), lambda i: (0, 0)),
        out_shape=jax.ShapeDtypeStruct((NG, 1), _F32),
        scratch_shapes=[pltpu.VMEM((NG, H), _F32),
                        pltpu.VMEM((NG, H), _F32),
                        pltpu.VMEM((NG, 1), _F32)],
    )(z3, s3, q3, g3, be3, batch_col, P1W, P1b, P2W, P2b, P3W, P3b)


def kernel(x, edge_index, batch, W_in, b_in, g0, be0, W1, b1, g1, be1,
           W2, b2, g2, be2, W3, b3, g3, be3, P1W, P1b, P2W, P2b, P3W, P3b):
    src = edge_index[0]
    dst = edge_index[1]
    zerosC = jnp.zeros((N, CH), _F32)
    onesC = jnp.ones((DEG_EB, CH), _F32)
    degp = _get_deg_sc()(dst, zerosC, onesC)
    dinv = _dinv(degp)
    agg2 = _get_agg(2)
    agg4 = _get_agg(4)

    z0, s0, q0 = _mm0(x, W_in, b_in.reshape(1, -1))
    p0 = _apply(z0, s0, q0, g0.reshape(1, -1), be0.reshape(1, -1), dinv, 2)
    a1 = agg2(src, dst, *p0)

    z1, s1, q1 = _mm_agg(a1, dinv, W1, b1.reshape(1, -1))
    p1 = _apply(z1, s1, q1, g1.reshape(1, -1), be1.reshape(1, -1), dinv, 2)
    a2 = agg2(src, dst, *p1)

    z2, s2, q2 = _mm_agg(a2, dinv, W2, b2.reshape(1, -1))
    p2 = _apply(z2, s2, q2, g2.reshape(1, -1), be2.reshape(1, -1), dinv, 4)
    a3 = agg4(src, dst, *p2)

    z3, s3, q3 = _mm_agg(a3, dinv, W3, b3.reshape(1, -1))
    out = _pool(z3, s3, q3, g3.reshape(1, -1), be3.reshape(1, -1),
                batch.reshape(N, 1), P1W, P1b.reshape(1, -1),
                P2W, P2b.reshape(1, -1), P3W, P3b.reshape(1, -1))
    return out[:, 0]


# speedup vs baseline: 10.3363x; 1.5144x over previous
import functools

import jax
import jax.numpy as jnp
from jax import lax
from jax.experimental import pallas as pl
from jax.experimental.pallas import tpu as pltpu
from jax.experimental.pallas import tpu_sc as plsc

N = 10000
E = 160000
NG = 64
CH = 128
RB = 1000
NB = N // RB
NSC = 16
SUB_STRIDE = 624
SUB_LEN = 640
EDGES_PER_SUB = E // NSC
EB = 80
NBLK = EDGES_PER_SUB // EB
SB = 25
NSB = NBLK // SB
SPAIR = (SB - 1) // 2
DEG_EB = 40
EDGES_PER_WORKER = E // 32
DEG_NBLK = EDGES_PER_WORKER // DEG_EB

_F32 = jnp.float32
_PREC = lax.Precision.DEFAULT

_sc_mesh = dict(core_axis_name="c", subcore_axis_name="s",
                num_cores=2, num_subcores=NSC)


_SC_CACHE = {}


def _get_deg_sc():
    if "deg" in _SC_CACHE:
        return _SC_CACHE["deg"]

    @functools.partial(
        pl.kernel,
        out_type=jax.ShapeDtypeStruct((2, N, CH), _F32),
        mesh=plsc.VectorSubcoreMesh(**_sc_mesh),
        scratch_types=[
            pltpu.VMEM((DEG_NBLK, DEG_EB), jnp.int32),
            pltpu.VMEM((DEG_EB, CH), _F32),
            pltpu.VMEM_SHARED((N, CH), _F32),
            pltpu.SemaphoreType.DMA,
            pltpu.SemaphoreType.DMA,
        ],
    )
    def _deg_sc(dst3_hbm, zeros_hbm, ones_hbm, out_hbm,
                didx, ones_v, deg_sh, sem0, sem1):
        c = lax.axis_index("c")
        s = lax.axis_index("s")
        rs = pl.ds(s * SUB_STRIDE, SUB_LEN)
        pltpu.sync_copy(zeros_hbm.at[rs], deg_sh.at[rs])
        pltpu.sync_copy(ones_hbm, ones_v)
        pltpu.sync_copy(dst3_hbm.at[c * NSC + s], didx)
        plsc.subcore_barrier()

        def scatter(i, sem):
            pltpu.async_copy(ones_v, deg_sh.at[didx.at[i]], sem, add=True)

        def wait_scatter(sem):
            pltpu.make_async_copy(
                ones_v, deg_sh.at[pl.ds(0, DEG_EB)], sem).wait()

        scatter(0, sem0)
        scatter(1, sem1)

        def body(k, carry):
            wait_scatter(sem0)
            scatter(2 * k, sem0)
            wait_scatter(sem1)
            scatter(2 * k + 1, sem1)
            return carry

        lax.fori_loop(1, (DEG_NBLK - 1) // 2, body, 0)
        wait_scatter(sem0)
        scatter(DEG_NBLK - 1, sem0)
        wait_scatter(sem0)
        wait_scatter(sem1)
        plsc.subcore_barrier()
        pltpu.sync_copy(deg_sh.at[rs], out_hbm.at[c, rs])

    _SC_CACHE["deg"] = _deg_sc
    return _deg_sc


def _get_agg(nchunk):
    key = ("agg", nchunk)
    if key in _SC_CACHE:
        return _SC_CACHE[key]
    outs = [jax.ShapeDtypeStruct((N, CH), _F32) for _ in range(nchunk)]
    scratch = [
        pltpu.VMEM((SB, EB), jnp.int32),
        pltpu.VMEM((SB, EB), jnp.int32),
        pltpu.VMEM((EB, CH), _F32),
        pltpu.VMEM((EB, CH), _F32),
        pltpu.VMEM_SHARED((N, CH), _F32),
        pltpu.SemaphoreType.DMA,
        pltpu.SemaphoreType.DMA,
        pltpu.SemaphoreType.DMA,
        pltpu.SemaphoreType.DMA,
    ]

    @functools.partial(
        pl.kernel,
        out_type=outs,
        mesh=plsc.VectorSubcoreMesh(**_sc_mesh),
        scratch_types=scratch,
    )
    def agg(src4_hbm, dst4_hbm, *rest):
        ps = rest[:nchunk]
        os_ = rest[nchunk:2 * nchunk]
        sidx, didx, rowsA, rowsB, acc, gsA, gsB, ssA, ssB = rest[2 * nchunk:]
        c = lax.axis_index("c")
        s = lax.axis_index("s")
        rs = pl.ds(s * SUB_STRIDE, SUB_LEN)
        for j in range(nchunk):
            p, o = ps[j], os_[j]

            @pl.when(c == (j % 2))
            def _process(p=p, o=o):
                pltpu.sync_copy(p.at[rs], acc.at[rs])
                plsc.subcore_barrier()

                def gather(i, buf, sem):
                    pltpu.async_copy(p.at[sidx.at[i]], buf, sem)

                def wait_gather(buf, sem):
                    pltpu.make_async_copy(p.at[pl.ds(0, EB)], buf, sem).wait()

                def scatter(i, buf, sem):
                    pltpu.async_copy(buf, acc.at[didx.at[i]], sem, add=True)

                def wait_scatter(buf, sem):
                    pltpu.make_async_copy(
                        buf, acc.at[pl.ds(0, EB)], sem).wait()

                def sb_body(sb, carry):
                    pltpu.sync_copy(src4_hbm.at[s, sb], sidx)
                    pltpu.sync_copy(dst4_hbm.at[s, sb], didx)
                    gather(0, rowsA, gsA)

                    def body(k, carry2):
                        i = 2 * k
                        wait_gather(rowsA, gsA)

                        @pl.when(k > 0)
                        def _():
                            wait_scatter(rowsB, ssB)

                        gather(i + 1, rowsB, gsB)
                        scatter(i, rowsA, ssA)
                        wait_gather(rowsB, gsB)
                        wait_scatter(rowsA, ssA)
                        gather(i + 2, rowsA, gsA)
                        scatter(i + 1, rowsB, ssB)
                        return carry2

                    lax.fori_loop(0, SPAIR, body, 0)
                    wait_gather(rowsA, gsA)
                    scatter(SB - 1, rowsA, ssA)
                    wait_scatter(rowsB, ssB)
                    wait_scatter(rowsA, ssA)
                    return carry

                lax.fori_loop(0, NSB, sb_body, 0)
                plsc.subcore_barrier()
                pltpu.sync_copy(acc.at[rs], o.at[rs])
                plsc.subcore_barrier()

    _SC_CACHE[key] = agg
    return agg


def _dinv(degp):
    def body(d_ref, o_ref):
        deg = 1.0 + d_ref[0, :, 0:1] + d_ref[1, :, 0:1]
        o_ref[...] = lax.rsqrt(deg)

    return pl.pallas_call(
        body,
        grid=(NB,),
        in_specs=[pl.BlockSpec((2, RB, CH), lambda i: (0, i, 0))],
        out_specs=pl.BlockSpec((RB, 1), lambda i: (i, 0)),
        out_shape=jax.ShapeDtypeStruct((N, 1), _F32),
    )(degp)


def _tree_colsum(z):
    parts = [jnp.sum(z[k * 125:(k + 1) * 125, :], 0, keepdims=True)
             for k in range(8)]
    return (((parts[0] + parts[1]) + (parts[2] + parts[3]))
            + ((parts[4] + parts[5]) + (parts[6] + parts[7])))


def _block_stats(z, i, mstack, qstack):
    mb = _tree_colsum(z) / RB
    dz = z - mb
    qb = _tree_colsum(dz * dz)
    mstack[pl.ds(i, 1), :] = mb
    qstack[pl.ds(i, 1), :] = qb


def _final_stats(mstack, qstack, m_ref, v_ref):
    mb = mstack[...]
    m = jnp.sum(mb, 0, keepdims=True) / NB
    dm = mb - m
    v = (jnp.sum(qstack[...], 0, keepdims=True)
         + RB * jnp.sum(dm * dm, 0, keepdims=True)) / N
    m_ref[...] = m
    v_ref[...] = v


def _mm0(x, W, b):
    K, H = W.shape

    def body(x_ref, w_ref, b_ref, z_ref, m_ref, v_ref, mstack, qstack):
        i = pl.program_id(0)
        z = lax.dot_general(x_ref[...], w_ref[...], (((1,), (0,)), ((), ())),
                            precision=_PREC, preferred_element_type=_F32)
        z = z + b_ref[...]
        z_ref[...] = z
        _block_stats(z, i, mstack, qstack)

        @pl.when(i == NB - 1)
        def _():
            _final_stats(mstack, qstack, m_ref, v_ref)

    return pl.pallas_call(
        body,
        grid=(NB,),
        in_specs=[pl.BlockSpec((RB, K), lambda i: (i, 0)),
                  pl.BlockSpec((K, H), lambda i: (0, 0)),
                  pl.BlockSpec((1, H), lambda i: (0, 0))],
        out_specs=[pl.BlockSpec((RB, H), lambda i: (i, 0)),
                   pl.BlockSpec((1, H), lambda i: (0, 0)),
                   pl.BlockSpec((1, H), lambda i: (0, 0))],
        out_shape=[jax.ShapeDtypeStruct((N, H), _F32),
                   jax.ShapeDtypeStruct((1, H), _F32),
                   jax.ShapeDtypeStruct((1, H), _F32)],
        scratch_shapes=[pltpu.VMEM((NB, H), _F32), pltpu.VMEM((NB, H), _F32)],
    )(x, W, b)


def _bn_mm(z, ssum, ssq, gamma, beta, W, dinv_col):
    Hin, Hout = W.shape
    nchunk = Hout // CH

    def body(z_ref, s_ref, q_ref, g_ref, be_ref, w_ref, dinv_ref, *outs):
        m = s_ref[...]
        v = q_ref[...]
        y = (z_ref[...] - m) * lax.rsqrt(v + 1e-5) * g_ref[...] + be_ref[...]
        h = jnp.maximum(y, 0.0)
        t = lax.dot_general(h, w_ref[...], (((1,), (0,)), ((), ())),
                            precision=_PREC, preferred_element_type=_F32)
        p = t * dinv_ref[...]
        for c in range(nchunk):
            outs[c][...] = p[:, c * CH:(c + 1) * CH]

    return pl.pallas_call(
        body,
        grid=(NB,),
        in_specs=[pl.BlockSpec((RB, Hin), lambda i: (i, 0)),
                  pl.BlockSpec((1, Hin), lambda i: (0, 0)),
                  pl.BlockSpec((1, Hin), lambda i: (0, 0)),
                  pl.BlockSpec((1, Hin), lambda i: (0, 0)),
                  pl.BlockSpec((1, Hin), lambda i: (0, 0)),
                  pl.BlockSpec((Hin, Hout), lambda i: (0, 0)),
                  pl.BlockSpec((RB, 1), lambda i: (i, 0))],
        out_specs=[pl.BlockSpec((RB, CH), lambda i: (i, 0))] * nchunk,
        out_shape=[jax.ShapeDtypeStruct((N, CH), _F32)] * nchunk,
    )(z, ssum, ssq, gamma, beta, W, dinv_col)


def _post(a_list, dinv_col, b):
    nchunk = len(a_list)
    H = nchunk * CH

    def body(*refs):
        a_refs = refs[:nchunk]
        dinv_ref, b_ref = refs[nchunk:nchunk + 2]
        z_ref, m_ref, v_ref, mstack, qstack = refs[nchunk + 2:]
        i = pl.program_id(0)
        dinv = dinv_ref[...]
        z = jnp.concatenate([a_refs[c][...] * dinv for c in range(nchunk)],
                            axis=1) + b_ref[...]
        z_ref[...] = z
        _block_stats(z, i, mstack, qstack)

        @pl.when(i == NB - 1)
        def _():
            _final_stats(mstack, qstack, m_ref, v_ref)

    return pl.pallas_call(
        body,
        grid=(NB,),
        in_specs=[pl.BlockSpec((RB, CH), lambda i: (i, 0))
                  for _ in range(nchunk)] +
                 [pl.BlockSpec((RB, 1), lambda i: (i, 0)),
                  pl.BlockSpec((1, H), lambda i: (0, 0))],
        out_specs=[pl.BlockSpec((RB, H), lambda i: (i, 0)),
                   pl.BlockSpec((1, H), lambda i: (0, 0)),
                   pl.BlockSpec((1, H), lambda i: (0, 0))],
        out_shape=[jax.ShapeDtypeStruct((N, H), _F32),
                   jax.ShapeDtypeStruct((1, H), _F32),
                   jax.ShapeDtypeStruct((1, H), _F32)],
        scratch_shapes=[pltpu.VMEM((NB, H), _F32), pltpu.VMEM((NB, H), _F32)],
    )(*a_list, dinv_col, b)


def _pool(z3, s3, q3, g3, be3, batch_col, P1W, P1b, P2W, P2b, P3W, P3b):
    H = z3.shape[1]

    def body(z_ref, s_ref, q_ref, g_ref, be_ref, b_ref,
             p1w, p1b, p2w, p2b, p3w, p3b, out_ref, ssum, smax, scnt):
        i = pl.program_id(0)

        @pl.when(i == 0)
        def _():
            ssum[...] = jnp.zeros_like(ssum)
            scnt[...] = jnp.zeros_like(scnt)
            smax[...] = jnp.full_like(smax, -jnp.inf)

        m = s_ref[...]
        v = q_ref[...]
        h = jnp.maximum(
            (z_ref[...] - m) * lax.rsqrt(v + 1e-5) * g_ref[...] + be_ref[...],
            0.0)
        btc = b_ref[...]
        g_lo = jnp.min(btc)
        g_hi = jnp.max(btc)

        def mbody(g, carry):
            mask = btc == g
            cur = jnp.max(jnp.where(mask, h, -jnp.inf), axis=0, keepdims=True)
            smax[pl.ds(g, 1), :] = jnp.maximum(smax[pl.ds(g, 1), :], cur)
            ssum[pl.ds(g, 1), :] += jnp.sum(
                jnp.where(mask, h, 0.0), axis=0, keepdims=True)
            scnt[pl.ds(g, 1), :] += jnp.sum(mask.astype(_F32))
            return carry

        lax.fori_loop(g_lo, g_hi + 1, mbody, 0)

        @pl.when(i == NB - 1)
        def _():
            s = ssum[...]
            cnt = jnp.maximum(scnt[...], 1.0)
            mean = s / cnt
            pooled = jnp.concatenate([mean, smax[...], s], axis=1)
            h1 = jnp.maximum(
                lax.dot_general(pooled, p1w[...], (((1,), (0,)), ((), ())),
                                precision=_PREC, preferred_element_type=_F32)
                + p1b[...], 0.0)
            h2 = jnp.maximum(
                lax.dot_general(h1, p2w[...], (((1,), (0,)), ((), ())),
                                precision=_PREC, preferred_element_type=_F32)
                + p2b[...], 0.0)
            out_ref[...] = lax.dot_general(
                h2, p3w[...], (((1,), (0,)), ((), ())),
                precision=_PREC, preferred_element_type=_F32) + p3b[...]

    return pl.pallas_call(
        body,
        grid=(NB,),
        in_specs=[pl.BlockSpec((RB, H), lambda i: (i, 0)),
                  pl.BlockSpec((1, H), lambda i: (0, 0)),
                  pl.BlockSpec((1, H), lambda i: (0, 0)),
                  pl.BlockSpec((1, H), lambda i: (0, 0)),
                  pl.BlockSpec((1, H), lambda i: (0, 0)),
                  pl.BlockSpec((RB, 1), lambda i: (i, 0)),
                  pl.BlockSpec(P1W.shape, lambda i: (0, 0)),
                  pl.BlockSpec((1, P1W.shape[1]), lambda i: (0, 0)),
                  pl.BlockSpec(P2W.shape, lambda i: (0, 0)),
                  pl.BlockSpec((1, P2W.shape[1]), lambda i: (0, 0)),
                  pl.BlockSpec(P3W.shape, lambda i: (0, 0)),
                  pl.BlockSpec((1, 1), lambda i: (0, 0))],
        out_specs=pl.BlockSpec((NG, 1), lambda i: (0, 0)),
        out_shape=jax.ShapeDtypeStruct((NG, 1), _F32),
        scratch_shapes=[pltpu.VMEM((NG, H), _F32),
                        pltpu.VMEM((NG, H), _F32),
                        pltpu.VMEM((NG, 1), _F32)],
    )(z3, s3, q3, g3, be3, batch_col, P1W, P1b, P2W, P2b, P3W, P3b)


def kernel(x, edge_index, batch, W_in, b_in, g0, be0, W1, b1, g1, be1,
           W2, b2, g2, be2, W3, b3, g3, be3, P1W, P1b, P2W, P2b, P3W, P3b):
    src3 = edge_index[0].reshape(NSC, NSB, SB, EB)
    dst3 = edge_index[1].reshape(NSC, NSB, SB, EB)
    dstd3 = edge_index[1].reshape(32, DEG_NBLK, DEG_EB)
    zerosC = jnp.zeros((N, CH), _F32)
    onesC = jnp.ones((DEG_EB, CH), _F32)
    degp = _get_deg_sc()(dstd3, zerosC, onesC)
    dinv = _dinv(degp)
    agg2 = _get_agg(2)
    agg4 = _get_agg(4)

    z0, s0, q0 = _mm0(x, W_in, b_in.reshape(1, -1))
    p1 = _bn_mm(z0, s0, q0, g0.reshape(1, -1), be0.reshape(1, -1), W1, dinv)
    a1 = agg2(src3, dst3, *p1)
    z1, s1, q1 = _post(a1, dinv, b1.reshape(1, -1))

    p2 = _bn_mm(z1, s1, q1, g1.reshape(1, -1), be1.reshape(1, -1), W2, dinv)
    a2 = agg4(src3, dst3, *p2)
    z2, s2, q2 = _post(a2, dinv, b2.reshape(1, -1))

    p3 = _bn_mm(z2, s2, q2, g2.reshape(1, -1), be2.reshape(1, -1), W3, dinv)
    a3 = agg4(src3, dst3, *p3)
    z3, s3, q3 = _post(a3, dinv, b3.reshape(1, -1))
    out = _pool(z3, s3, q3, g3.reshape(1, -1), be3.reshape(1, -1),
                batch.reshape(N, 1), P1W, P1b.reshape(1, -1),
                P2W, P2b.reshape(1, -1), P3W, P3b.reshape(1, -1))
    return out[:, 0]
